# baseline (speedup 1.0000x reference)
import functools

import jax
import jax.numpy as jnp
from jax.experimental import pallas as pl
from jax.experimental.pallas import tpu as pltpu

E = 8
EPAD = 16
NEXP = 9
T = 2048
D = 768
H = 512
COEF = 0.01
NEG = -1e30


def _router_body(x_ref, wr_ref, comb_ref, aux_ref):
    x = x_ref[...]
    wr = wr_ref[...]
    logits = jax.lax.dot_general(x, wr, (((1,), (1,)), ((), ())),
                                 preferred_element_type=jnp.float32)
    lane = jax.lax.broadcasted_iota(jnp.int32, (T, EPAD), 1)
    valid = lane < E
    logits = jnp.where(valid, logits, NEG)

    m = jnp.max(logits, axis=1, keepdims=True)
    ex = jnp.exp(logits - m)
    ex = jnp.where(valid, ex, 0.0)
    probs = ex / jnp.sum(ex, axis=1, keepdims=True)

    i1 = jnp.argmax(logits, axis=1).reshape(T, 1)
    oh1 = (lane == i1).astype(jnp.float32)
    p1 = jnp.max(probs, axis=1, keepdims=True)
    logits2 = jnp.where(lane == i1, NEG, logits)
    i2 = jnp.argmax(logits2, axis=1).reshape(T, 1)
    oh2 = (lane == i2).astype(jnp.float32)
    p2 = jnp.max(jnp.where(lane == i1, NEG, probs), axis=1, keepdims=True)

    comb = p1 * oh1 + p2 * oh2 + (lane == E).astype(jnp.float32)
    comb_ref[...] = comb

    density = jnp.mean(oh1, axis=0, keepdims=True)
    rpm = jnp.mean(probs, axis=0, keepdims=True)
    aux = COEF * jnp.sum(density * rpm) * E
    aux_ref[...] = jnp.full((8, 128), aux, dtype=jnp.float32)


def _expert_body(x_ref, comb_ref, w1_ref, w3_ref, w2_ref, out_ref):
    e = pl.program_id(0)
    x = x_ref[...]
    w1 = w1_ref[0]
    w3 = w3_ref[0]
    w2 = w2_ref[0]
    h1 = jax.lax.dot_general(x, w1, (((1,), (1,)), ((), ())),
                             preferred_element_type=jnp.float32)
    h3 = jax.lax.dot_general(x, w3, (((1,), (1,)), ((), ())),
                             preferred_element_type=jnp.float32)
    g = (h1 * jax.nn.sigmoid(h1) * h3).astype(jnp.bfloat16)
    o = jax.lax.dot_general(g, w2, (((1,), (1,)), ((), ())),
                            preferred_element_type=jnp.float32)
    lane = jax.lax.broadcasted_iota(jnp.int32, (T, EPAD), 1)
    w = jnp.sum(jnp.where(lane == e, comb_ref[...], 0.0), axis=1, keepdims=True)
    contrib = o * w

    @pl.when(e == 0)
    def _():
        out_ref[...] = contrib

    @pl.when(e != 0)
    def _():
        out_ref[...] += contrib


@functools.partial(jax.jit, static_argnames=())
def kernel(x, Wr, W1, W2, W3, sW1, sW2, sW3):
    B, S, Dm = x.shape
    x_flat = x.reshape(T, D)
    wr_pad = jnp.zeros((EPAD, D), jnp.float32).at[:E].set(Wr)
    x_bf = x_flat.astype(jnp.bfloat16)
    w1c = jnp.concatenate([W1, sW1], axis=0).astype(jnp.bfloat16)
    w3c = jnp.concatenate([W3, sW3], axis=0).astype(jnp.bfloat16)
    w2c = jnp.concatenate([W2, sW2], axis=0).astype(jnp.bfloat16)

    comb, aux = pl.pallas_call(
        _router_body,
        out_shape=(
            jax.ShapeDtypeStruct((T, EPAD), jnp.float32),
            jax.ShapeDtypeStruct((8, 128), jnp.float32),
        ),
    )(x_flat, wr_pad)

    out = pl.pallas_call(
        _expert_body,
        grid=(NEXP,),
        in_specs=[
            pl.BlockSpec((T, D), lambda e: (0, 0)),
            pl.BlockSpec((T, EPAD), lambda e: (0, 0)),
            pl.BlockSpec((1, H, D), lambda e: (e, 0, 0)),
            pl.BlockSpec((1, H, D), lambda e: (e, 0, 0)),
            pl.BlockSpec((1, D, H), lambda e: (e, 0, 0)),
        ],
        out_specs=pl.BlockSpec((T, D), lambda e: (0, 0)),
        out_shape=jax.ShapeDtypeStruct((T, D), jnp.float32),
    )(x_bf, comb, w1c, w3c, w2c)

    return out.reshape(B, S, Dm), aux[0, 0]


# speedup vs baseline: 1.0455x; 1.0455x over previous
import functools

import jax
import jax.numpy as jnp
from jax.experimental import pallas as pl
from jax.experimental.pallas import tpu as pltpu

E = 8
EPAD = 16
NEXP = 9
T = 2048
D = 768
H = 512
COEF = 0.01
NEG = -1e30


def _router_body(x_ref, wr_ref, comb_ref, aux_ref):
    x = x_ref[...]
    wr = wr_ref[...]
    logits = jax.lax.dot_general(x, wr, (((1,), (1,)), ((), ())),
                                 preferred_element_type=jnp.float32)
    lane = jax.lax.broadcasted_iota(jnp.int32, (T, EPAD), 1)
    valid = lane < E
    logits = jnp.where(valid, logits, NEG)

    m = jnp.max(logits, axis=1, keepdims=True)
    ex = jnp.exp(logits - m)
    ex = jnp.where(valid, ex, 0.0)
    probs = ex / jnp.sum(ex, axis=1, keepdims=True)

    i1 = jnp.argmax(logits, axis=1).reshape(T, 1)
    oh1 = (lane == i1).astype(jnp.float32)
    p1 = jnp.max(probs, axis=1, keepdims=True)
    logits2 = jnp.where(lane == i1, NEG, logits)
    i2 = jnp.argmax(logits2, axis=1).reshape(T, 1)
    oh2 = (lane == i2).astype(jnp.float32)
    p2 = jnp.max(jnp.where(lane == i1, NEG, probs), axis=1, keepdims=True)

    comb = p1 * oh1 + p2 * oh2 + (lane == E).astype(jnp.float32)
    comb_ref[...] = comb

    density = jnp.mean(oh1, axis=0, keepdims=True)
    rpm = jnp.mean(probs, axis=0, keepdims=True)
    aux = COEF * jnp.sum(density * rpm) * E
    aux_ref[...] = jnp.full((8, 128), aux, dtype=jnp.float32)


def _expert_body(x_ref, comb_ref, w1_ref, w3_ref, w2_ref, out_ref):
    e = pl.program_id(0)
    x = x_ref[...]
    w1 = w1_ref[0]
    w3 = w3_ref[0]
    w2 = w2_ref[0]
    h1 = jax.lax.dot_general(x, w1, (((1,), (1,)), ((), ())),
                             preferred_element_type=jnp.float32)
    h3 = jax.lax.dot_general(x, w3, (((1,), (1,)), ((), ())),
                             preferred_element_type=jnp.float32)
    g = h1 * jax.nn.sigmoid(h1) * h3
    o = jax.lax.dot_general(g, w2, (((1,), (1,)), ((), ())),
                            preferred_element_type=jnp.float32)
    lane = jax.lax.broadcasted_iota(jnp.int32, (T, EPAD), 1)
    w = jnp.sum(jnp.where(lane == e, comb_ref[...], 0.0), axis=1, keepdims=True)
    contrib = o * w

    @pl.when(e == 0)
    def _():
        out_ref[...] = contrib

    @pl.when(e != 0)
    def _():
        out_ref[...] += contrib


@functools.partial(jax.jit, static_argnames=())
def kernel(x, Wr, W1, W2, W3, sW1, sW2, sW3):
    B, S, Dm = x.shape
    x_flat = x.reshape(T, D)
    wr_pad = jnp.zeros((EPAD, D), jnp.float32).at[:E].set(Wr)
    w1c = jnp.concatenate([W1, sW1], axis=0)
    w3c = jnp.concatenate([W3, sW3], axis=0)
    w2c = jnp.concatenate([W2, sW2], axis=0)

    comb, aux = pl.pallas_call(
        _router_body,
        out_shape=(
            jax.ShapeDtypeStruct((T, EPAD), jnp.float32),
            jax.ShapeDtypeStruct((8, 128), jnp.float32),
        ),
    )(x_flat, wr_pad)

    out = pl.pallas_call(
        _expert_body,
        grid=(NEXP,),
        in_specs=[
            pl.BlockSpec((T, D), lambda e: (0, 0)),
            pl.BlockSpec((T, EPAD), lambda e: (0, 0)),
            pl.BlockSpec((1, H, D), lambda e: (e, 0, 0)),
            pl.BlockSpec((1, H, D), lambda e: (e, 0, 0)),
            pl.BlockSpec((1, D, H), lambda e: (e, 0, 0)),
        ],
        out_specs=pl.BlockSpec((T, D), lambda e: (0, 0)),
        out_shape=jax.ShapeDtypeStruct((T, D), jnp.float32),
    )(x_flat, comb, w1c, w3c, w2c)

    return out.reshape(B, S, Dm), aux[0, 0]


# speedup vs baseline: 1.3998x; 1.3389x over previous
import jax
import jax.numpy as jnp
from jax.experimental import pallas as pl

E = 8
EPAD = 16
NEXP = 9
T = 2048
D = 768
H = 512
COEF = 0.01
NEG = -1e30


def _router_body(x_ref, wr_ref, comb_ref, aux_ref):
    x = x_ref[...]
    wr = wr_ref[...]
    logits = jax.lax.dot_general(x, wr, (((1,), (1,)), ((), ())),
                                 preferred_element_type=jnp.float32)
    lane = jax.lax.broadcasted_iota(jnp.int32, (T, EPAD), 1)
    valid = lane < E
    logits = jnp.where(valid, logits, NEG)

    m = jnp.max(logits, axis=1, keepdims=True)
    ex = jnp.exp(logits - m)
    ex = jnp.where(valid, ex, 0.0)
    probs = ex / jnp.sum(ex, axis=1, keepdims=True)

    i1 = jnp.argmax(logits, axis=1).reshape(T, 1)
    oh1 = (lane == i1).astype(jnp.float32)
    p1 = jnp.max(probs, axis=1, keepdims=True)
    logits2 = jnp.where(lane == i1, NEG, logits)
    i2 = jnp.argmax(logits2, axis=1).reshape(T, 1)
    oh2 = (lane == i2).astype(jnp.float32)
    p2 = jnp.max(jnp.where(lane == i1, NEG, probs), axis=1, keepdims=True)

    comb = p1 * oh1 + p2 * oh2 + (lane == E).astype(jnp.float32)
    comb_ref[...] = comb

    density = jnp.mean(oh1, axis=0, keepdims=True)
    rpm = jnp.mean(probs, axis=0, keepdims=True)
    aux = COEF * jnp.sum(density * rpm) * E
    aux_ref[...] = jnp.full((8, 128), aux, dtype=jnp.float32)


def _mlp(x, w1, w3, w2):
    h1 = jax.lax.dot_general(x, w1, (((1,), (1,)), ((), ())),
                             preferred_element_type=jnp.float32)
    h3 = jax.lax.dot_general(x, w3, (((1,), (1,)), ((), ())),
                             preferred_element_type=jnp.float32)
    g = h1 * jax.nn.sigmoid(h1) * h3
    return jax.lax.dot_general(g, w2, (((1,), (1,)), ((), ())),
                               preferred_element_type=jnp.float32)


def _expert_body(x_ref, comb_ref, w1_ref, w3_ref, w2_ref,
                 sw1_ref, sw3_ref, sw2_ref, out_ref):
    e = pl.program_id(0)
    x = x_ref[...]

    @pl.when(e < E)
    def _():
        o = _mlp(x, w1_ref[0], w3_ref[0], w2_ref[0])
        lane = jax.lax.broadcasted_iota(jnp.int32, (T, EPAD), 1)
        w = jnp.sum(jnp.where(lane == e, comb_ref[...], 0.0),
                    axis=1, keepdims=True)
        contrib = o * w

        @pl.when(e == 0)
        def _():
            out_ref[...] = contrib

        @pl.when(e != 0)
        def _():
            out_ref[...] += contrib

    @pl.when(e == E)
    def _():
        out_ref[...] += _mlp(x, sw1_ref[0], sw3_ref[0], sw2_ref[0])


def kernel(x, Wr, W1, W2, W3, sW1, sW2, sW3):
    B, S, Dm = x.shape
    x_flat = x.reshape(T, D)
    wr_pad = jnp.zeros((EPAD, D), jnp.float32).at[:E].set(Wr)

    comb, aux = pl.pallas_call(
        _router_body,
        out_shape=(
            jax.ShapeDtypeStruct((T, EPAD), jnp.float32),
            jax.ShapeDtypeStruct((8, 128), jnp.float32),
        ),
    )(x_flat, wr_pad)

    out = pl.pallas_call(
        _expert_body,
        grid=(NEXP,),
        in_specs=[
            pl.BlockSpec((T, D), lambda e: (0, 0)),
            pl.BlockSpec((T, EPAD), lambda e: (0, 0)),
            pl.BlockSpec((1, H, D), lambda e: (jnp.minimum(e, E - 1), 0, 0)),
            pl.BlockSpec((1, H, D), lambda e: (jnp.minimum(e, E - 1), 0, 0)),
            pl.BlockSpec((1, D, H), lambda e: (jnp.minimum(e, E - 1), 0, 0)),
            pl.BlockSpec((1, H, D), lambda e: (0, 0, 0)),
            pl.BlockSpec((1, H, D), lambda e: (0, 0, 0)),
            pl.BlockSpec((1, D, H), lambda e: (0, 0, 0)),
        ],
        out_specs=pl.BlockSpec((T, D), lambda e: (0, 0)),
        out_shape=jax.ShapeDtypeStruct((T, D), jnp.float32),
    )(x_flat, comb, W1, W3, W2, sW1, sW3, sW2)

    return out.reshape(B, S, Dm), aux[0, 0]
